# baseline (speedup 1.0000x reference)
import functools

import jax
import jax.numpy as jnp
from jax import lax
from jax.experimental import pallas as pl
from jax.experimental.pallas import tpu as pltpu
from jax.experimental.pallas import tpu_sc as plsc

_NUM_WORKERS = 32
_BLK = 128
_D = 64


@jax.jit
def _embed(ids_t, table2):
    s, b_rows = ids_t.shape
    nblk = b_rows // _BLK
    n_units = s * nblk
    u_per_w = n_units // _NUM_WORKERS

    mesh = plsc.VectorSubcoreMesh(core_axis_name="c", subcore_axis_name="s")

    @functools.partial(
        pl.kernel,
        mesh=mesh,
        out_type=jax.ShapeDtypeStruct((s, _D, b_rows), jnp.float32),
        scratch_types=[
            pltpu.VMEM((2, _BLK), jnp.int32),
            pltpu.VMEM((2, _BLK), jnp.int32),
            pltpu.VMEM((2, _BLK, 128), jnp.float32),
            pltpu.VMEM((2, _D, _BLK), jnp.float32),
            pltpu.SemaphoreType.DMA,
            pltpu.SemaphoreType.DMA,
            pltpu.SemaphoreType.DMA,
            pltpu.SemaphoreType.DMA,
            pltpu.SemaphoreType.DMA,
            pltpu.SemaphoreType.DMA,
        ],
        compiler_params=pltpu.CompilerParams(needs_layout_passes=False),
    )
    def k(ids_hbm, tab_hbm, out_hbm, idx_v, ridx_v, pbuf, sbuf,
          si0, si1, sg0, sg1, so0, so1):
        sem_i = (si0, si1)
        sem_g = (sg0, sg1)
        sem_o = (so0, so1)
        wid = lax.axis_index("s") * 2 + lax.axis_index("c")
        u0 = wid * u_per_w
        iota16 = lax.iota(jnp.int32, 16)

        def idx_start(u, b):
            i1 = u // nblk
            c0 = (u % nblk) * _BLK
            pltpu.async_copy(ids_hbm.at[i1, pl.ds(c0, _BLK)], idx_v.at[b],
                             sem_i[b])

        def idx_wait(u, b):
            i1 = u // nblk
            c0 = (u % nblk) * _BLK
            pltpu.make_async_copy(ids_hbm.at[i1, pl.ds(c0, _BLK)],
                                  idx_v.at[b], sem_i[b]).wait()

        def ridx_compute(b):
            pass

        def gather_start(b):
            pltpu.async_copy(tab_hbm.at[idx_v.at[b]], pbuf.at[b], sem_g[b])

        def gather_wait(b):
            pltpu.make_async_copy(tab_hbm.at[idx_v.at[b]], pbuf.at[b],
                                  sem_g[b]).wait()

        rolls = [(iota16 + d) & 15 for d in range(16)]

        def select(b):
            def grp(g, carry):
                rows = lax.broadcast(g * 16, (16,)) + iota16
                for fb in range(_D // 16):
                    for d0 in range(0, 16, 16):
                        cs = [rolls[d0 + i] + (fb * 16) for i in range(16)]
                        vs = [plsc.load_gather(pbuf.at[b], [rows, c])
                              for c in cs]
                        for c, v in zip(cs, vs):
                            plsc.store_scatter(sbuf.at[b], [c, rows], v)
                return carry
            lax.fori_loop(0, _BLK // 16, grp, 0)

        def out_start(u, b):
            i1 = u // nblk
            c0 = (u % nblk) * _BLK
            pltpu.async_copy(sbuf.at[b], out_hbm.at[i1, :, pl.ds(c0, _BLK)],
                             sem_o[b])

        def out_wait(u, b):
            i1 = u // nblk
            c0 = (u % nblk) * _BLK
            pltpu.make_async_copy(sbuf.at[b],
                                  out_hbm.at[i1, :, pl.ds(c0, _BLK)],
                                  sem_o[b]).wait()

        idx_start(u0 + 0, 0)
        idx_wait(u0 + 0, 0)
        ridx_compute(0)
        gather_start(0)
        idx_start(u0 + 1, 1)

        def body2(t, carry):
            for b in range(2):
                u = t * 2 + 1 + b
                bf = 1 - b
                idx_wait(u0 + u, bf)
                ridx_compute(bf)
                gather_start(bf)
                gather_wait(b)

                @pl.when(u >= 3)
                def _():
                    out_wait(u0 + u - 3, b)
                select(b)
                out_start(u0 + u - 1, b)
                idx_start(u0 + u + 1, b)
            return carry

        lax.fori_loop(0, (u_per_w - 2) // 2, body2, 0)

        idx_wait(u0 + u_per_w - 1, 1)
        ridx_compute(1)
        gather_start(1)
        gather_wait(0)
        out_wait(u0 + u_per_w - 4, 0)
        select(0)
        out_start(u0 + u_per_w - 2, 0)
        gather_wait(1)
        out_wait(u0 + u_per_w - 3, 1)
        select(1)
        out_start(u0 + u_per_w - 1, 1)
        out_wait(u0 + u_per_w - 2, 0)
        out_wait(u0 + u_per_w - 1, 1)

    return k(ids_t, table2)


def kernel(ids, table):
    b, s = ids.shape
    d = table.shape[1]
    ids_t = jnp.swapaxes(ids, 0, 1).astype(jnp.int32)
    table2 = jnp.pad(table, ((0, 0), (0, 128 - d)))
    out_t = _embed(ids_t, table2)
    return jnp.transpose(out_t, (2, 0, 1))


# speedup vs baseline: 1.0558x; 1.0558x over previous
import functools

import jax
import jax.numpy as jnp
from jax import lax
from jax.experimental import pallas as pl
from jax.experimental.pallas import tpu as pltpu
from jax.experimental.pallas import tpu_sc as plsc

_NUM_WORKERS = 32
_BLK = 128
_D = 64


@jax.jit
def _embed(ids_t, table2):
    s, b_rows = ids_t.shape
    nblk = b_rows // _BLK
    n_units = s * nblk
    u_per_w = n_units // _NUM_WORKERS

    mesh = plsc.VectorSubcoreMesh(core_axis_name="c", subcore_axis_name="s")

    @functools.partial(
        pl.kernel,
        mesh=mesh,
        out_type=jax.ShapeDtypeStruct((s, _D, b_rows), jnp.float32),
        scratch_types=[
            pltpu.VMEM((2, _BLK), jnp.int32),
            pltpu.VMEM((2, _BLK), jnp.int32),
            pltpu.VMEM((2, _BLK, 128), jnp.float32),
            pltpu.VMEM((2, _D, _BLK), jnp.float32),
            pltpu.SemaphoreType.DMA,
            pltpu.SemaphoreType.DMA,
            pltpu.SemaphoreType.DMA,
            pltpu.SemaphoreType.DMA,
            pltpu.SemaphoreType.DMA,
            pltpu.SemaphoreType.DMA,
        ],
        compiler_params=pltpu.CompilerParams(needs_layout_passes=False),
    )
    def k(ids_hbm, tab_hbm, out_hbm, idx_v, ridx_v, pbuf, sbuf,
          si0, si1, sg0, sg1, so0, so1):
        sem_i = (si0, si1)
        sem_g = (sg0, sg1)
        sem_o = (so0, so1)
        wid = lax.axis_index("s") * 2 + lax.axis_index("c")
        u0 = wid * u_per_w
        iota16 = lax.iota(jnp.int32, 16)

        def idx_start(u, b):
            i1 = u // nblk
            c0 = (u % nblk) * _BLK
            pltpu.async_copy(ids_hbm.at[i1, pl.ds(c0, _BLK)], idx_v.at[b],
                             sem_i[b])

        def idx_wait(u, b):
            i1 = u // nblk
            c0 = (u % nblk) * _BLK
            pltpu.make_async_copy(ids_hbm.at[i1, pl.ds(c0, _BLK)],
                                  idx_v.at[b], sem_i[b]).wait()

        def ridx_compute(b):
            pass

        def gather_start(b):
            pltpu.async_copy(tab_hbm.at[idx_v.at[b]], pbuf.at[b], sem_g[b])

        def gather_wait(b):
            pltpu.make_async_copy(tab_hbm.at[idx_v.at[b]], pbuf.at[b],
                                  sem_g[b]).wait()

        rolls = [(iota16 + d) & 15 for d in range(16)]

        def select(b):
            def grp(g2, carry):
              for gi in range(2):
                g = g2 * 2 + gi
                rows = lax.broadcast(g * 16, (16,)) + iota16
                for fb in range(_D // 16):
                    for d0 in range(0, 16, 8):
                        cs = [rolls[d0 + i] + (fb * 16) for i in range(8)]
                        vs = [plsc.load_gather(pbuf.at[b], [rows, c])
                              for c in cs]
                        for c, v in zip(cs, vs):
                            plsc.store_scatter(sbuf.at[b], [c, rows], v)
              return carry
            lax.fori_loop(0, _BLK // 32, grp, 0)

        def out_start(u, b):
            i1 = u // nblk
            c0 = (u % nblk) * _BLK
            pltpu.async_copy(sbuf.at[b], out_hbm.at[i1, :, pl.ds(c0, _BLK)],
                             sem_o[b])

        def out_wait(u, b):
            i1 = u // nblk
            c0 = (u % nblk) * _BLK
            pltpu.make_async_copy(sbuf.at[b],
                                  out_hbm.at[i1, :, pl.ds(c0, _BLK)],
                                  sem_o[b]).wait()

        idx_start(u0 + 0, 0)
        idx_wait(u0 + 0, 0)
        ridx_compute(0)
        gather_start(0)
        idx_start(u0 + 1, 1)

        def body2(t, carry):
            for b in range(2):
                u = t * 2 + 1 + b
                bf = 1 - b
                idx_wait(u0 + u, bf)
                ridx_compute(bf)
                gather_start(bf)
                gather_wait(b)
                idx_start(u0 + u + 1, b)

                @pl.when(u >= 3)
                def _():
                    out_wait(u0 + u - 3, b)
                select(b)
                out_start(u0 + u - 1, b)
            return carry

        lax.fori_loop(0, (u_per_w - 2) // 2, body2, 0)

        idx_wait(u0 + u_per_w - 1, 1)
        ridx_compute(1)
        gather_start(1)
        gather_wait(0)
        out_wait(u0 + u_per_w - 4, 0)
        select(0)
        out_start(u0 + u_per_w - 2, 0)
        gather_wait(1)
        out_wait(u0 + u_per_w - 3, 1)
        select(1)
        out_start(u0 + u_per_w - 1, 1)
        out_wait(u0 + u_per_w - 2, 0)
        out_wait(u0 + u_per_w - 1, 1)

    return k(ids_t, table2)


def kernel(ids, table):
    b, s = ids.shape
    d = table.shape[1]
    ids_t = jnp.swapaxes(ids, 0, 1).astype(jnp.int32)
    table2 = jnp.pad(table, ((0, 0), (0, 128 - d)))
    out_t = _embed(ids_t, table2)
    return jnp.transpose(out_t, (2, 0, 1))


# speedup vs baseline: 1.0610x; 1.0049x over previous
import functools

import jax
import jax.numpy as jnp
from jax import lax
from jax.experimental import pallas as pl
from jax.experimental.pallas import tpu as pltpu
from jax.experimental.pallas import tpu_sc as plsc

_NUM_WORKERS = 32
_BLK = 128
_D = 64


@jax.jit
def _embed(ids_t, table2):
    s, b_rows = ids_t.shape
    nblk = b_rows // _BLK
    n_units = s * nblk
    u_per_w = n_units // _NUM_WORKERS

    mesh = plsc.VectorSubcoreMesh(core_axis_name="c", subcore_axis_name="s")

    @functools.partial(
        pl.kernel,
        mesh=mesh,
        out_type=jax.ShapeDtypeStruct((s, _D, b_rows), jnp.float32),
        scratch_types=[
            pltpu.VMEM((2, _BLK), jnp.int32),
            pltpu.VMEM((2, _BLK), jnp.int32),
            pltpu.VMEM((2, _BLK, 128), jnp.float32),
            pltpu.VMEM((2, _D, _BLK), jnp.float32),
            pltpu.SemaphoreType.DMA,
            pltpu.SemaphoreType.DMA,
            pltpu.SemaphoreType.DMA,
            pltpu.SemaphoreType.DMA,
            pltpu.SemaphoreType.DMA,
            pltpu.SemaphoreType.DMA,
        ],
        compiler_params=pltpu.CompilerParams(needs_layout_passes=False),
    )
    def k(ids_hbm, tab_hbm, out_hbm, idx_v, ridx_v, pbuf, sbuf,
          si0, si1, sg0, sg1, so0, so1):
        sem_i = (si0, si1)
        sem_g = (sg0, sg1)
        sem_o = (so0, so1)
        wid = lax.axis_index("s") * 2 + lax.axis_index("c")
        u0 = wid * u_per_w
        iota16 = lax.iota(jnp.int32, 16)

        def idx_start(u, b):
            i1 = u // nblk
            c0 = (u % nblk) * _BLK
            pltpu.async_copy(ids_hbm.at[i1, pl.ds(c0, _BLK)], idx_v.at[b],
                             sem_i[b])

        def idx_wait(u, b):
            i1 = u // nblk
            c0 = (u % nblk) * _BLK
            pltpu.make_async_copy(ids_hbm.at[i1, pl.ds(c0, _BLK)],
                                  idx_v.at[b], sem_i[b]).wait()

        def ridx_compute(b):
            pass

        def gather_start(b):
            pltpu.async_copy(tab_hbm.at[idx_v.at[b]], pbuf.at[b], sem_g[b])

        def gather_wait(b):
            pltpu.make_async_copy(tab_hbm.at[idx_v.at[b]], pbuf.at[b],
                                  sem_g[b]).wait()

        rolls = [(iota16 + d) & 15 for d in range(16)]

        def select(b):
            def grp(g2, carry):
              for gi in range(4):
                g = g2 * 4 + gi
                rows = lax.broadcast(g * 16, (16,)) + iota16
                for fb in range(_D // 16):
                    for d0 in range(0, 16, 8):
                        cs = [rolls[d0 + i] + (fb * 16) for i in range(8)]
                        vs = [plsc.load_gather(pbuf.at[b], [rows, c])
                              for c in cs]
                        for c, v in zip(cs, vs):
                            plsc.store_scatter(sbuf.at[b], [c, rows], v)
              return carry
            lax.fori_loop(0, _BLK // 64, grp, 0)

        def out_start(u, b):
            i1 = u // nblk
            c0 = (u % nblk) * _BLK
            pltpu.async_copy(sbuf.at[b], out_hbm.at[i1, :, pl.ds(c0, _BLK)],
                             sem_o[b])

        def out_wait(u, b):
            i1 = u // nblk
            c0 = (u % nblk) * _BLK
            pltpu.make_async_copy(sbuf.at[b],
                                  out_hbm.at[i1, :, pl.ds(c0, _BLK)],
                                  sem_o[b]).wait()

        idx_start(u0 + 0, 0)
        idx_wait(u0 + 0, 0)
        ridx_compute(0)
        gather_start(0)
        idx_start(u0 + 1, 1)

        def body2(t, carry):
            for b in range(2):
                u = t * 2 + 1 + b
                bf = 1 - b
                idx_wait(u0 + u, bf)
                ridx_compute(bf)
                gather_start(bf)
                gather_wait(b)
                idx_start(u0 + u + 1, b)

                @pl.when(u >= 3)
                def _():
                    out_wait(u0 + u - 3, b)
                select(b)
                out_start(u0 + u - 1, b)
            return carry

        lax.fori_loop(0, (u_per_w - 2) // 2, body2, 0)

        idx_wait(u0 + u_per_w - 1, 1)
        ridx_compute(1)
        gather_start(1)
        gather_wait(0)
        out_wait(u0 + u_per_w - 4, 0)
        select(0)
        out_start(u0 + u_per_w - 2, 0)
        gather_wait(1)
        out_wait(u0 + u_per_w - 3, 1)
        select(1)
        out_start(u0 + u_per_w - 1, 1)
        out_wait(u0 + u_per_w - 2, 0)
        out_wait(u0 + u_per_w - 1, 1)

    return k(ids_t, table2)


def kernel(ids, table):
    b, s = ids.shape
    d = table.shape[1]
    ids_t = jnp.swapaxes(ids, 0, 1).astype(jnp.int32)
    table2 = jnp.pad(table, ((0, 0), (0, 128 - d)))
    out_t = _embed(ids_t, table2)
    return jnp.transpose(out_t, (2, 0, 1))


# speedup vs baseline: 1.0614x; 1.0004x over previous
import functools

import jax
import jax.numpy as jnp
from jax import lax
from jax.experimental import pallas as pl
from jax.experimental.pallas import tpu as pltpu
from jax.experimental.pallas import tpu_sc as plsc

_NUM_WORKERS = 32
_BLK = 128
_D = 64


@jax.jit
def _embed(ids_t, table2):
    s, b_rows = ids_t.shape
    nblk = b_rows // _BLK
    n_units = s * nblk
    u_per_w = n_units // _NUM_WORKERS

    mesh = plsc.VectorSubcoreMesh(core_axis_name="c", subcore_axis_name="s")

    @functools.partial(
        pl.kernel,
        mesh=mesh,
        out_type=jax.ShapeDtypeStruct((s, _D, b_rows), jnp.float32),
        scratch_types=[
            pltpu.VMEM((2, _BLK), jnp.int32),
            pltpu.VMEM((2, _BLK, 128), jnp.float32),
            pltpu.VMEM((2, _D, _BLK), jnp.float32),
            pltpu.SemaphoreType.DMA,
            pltpu.SemaphoreType.DMA,
            pltpu.SemaphoreType.DMA,
            pltpu.SemaphoreType.DMA,
            pltpu.SemaphoreType.DMA,
            pltpu.SemaphoreType.DMA,
        ],
        compiler_params=pltpu.CompilerParams(needs_layout_passes=False),
    )
    def k(ids_hbm, tab_hbm, out_hbm, idx_v, pbuf, sbuf,
          si0, si1, sg0, sg1, so0, so1):
        sem_i = (si0, si1)
        sem_g = (sg0, sg1)
        sem_o = (so0, so1)
        wid = lax.axis_index("s") * 2 + lax.axis_index("c")
        u0 = wid * u_per_w
        iota16 = lax.iota(jnp.int32, 16)

        def idx_start(u, b):
            i1 = u // nblk
            c0 = (u % nblk) * _BLK
            pltpu.async_copy(ids_hbm.at[i1, pl.ds(c0, _BLK)], idx_v.at[b],
                             sem_i[b])

        def idx_wait(u, b):
            i1 = u // nblk
            c0 = (u % nblk) * _BLK
            pltpu.make_async_copy(ids_hbm.at[i1, pl.ds(c0, _BLK)],
                                  idx_v.at[b], sem_i[b]).wait()

        def gather_start(b):
            pltpu.async_copy(tab_hbm.at[idx_v.at[b]], pbuf.at[b], sem_g[b])

        def gather_wait(b):
            pltpu.make_async_copy(tab_hbm.at[idx_v.at[b]], pbuf.at[b],
                                  sem_g[b]).wait()

        rolls = [(iota16 + d) & 15 for d in range(16)]

        def select(b):
            def grp(g2, carry):
              for gi in range(4):
                g = g2 * 4 + gi
                rows = lax.broadcast(g * 16, (16,)) + iota16
                for fb in range(_D // 16):
                    for d0 in range(0, 16, 8):
                        cs = [rolls[d0 + i] + (fb * 16) for i in range(8)]
                        vs = [plsc.load_gather(pbuf.at[b], [rows, c])
                              for c in cs]
                        for c, v in zip(cs, vs):
                            plsc.store_scatter(sbuf.at[b], [c, rows], v)
              return carry
            lax.fori_loop(0, _BLK // 64, grp, 0)

        def out_start(u, b):
            i1 = u // nblk
            c0 = (u % nblk) * _BLK
            pltpu.async_copy(sbuf.at[b], out_hbm.at[i1, :, pl.ds(c0, _BLK)],
                             sem_o[b])

        def out_wait(u, b):
            i1 = u // nblk
            c0 = (u % nblk) * _BLK
            pltpu.make_async_copy(sbuf.at[b],
                                  out_hbm.at[i1, :, pl.ds(c0, _BLK)],
                                  sem_o[b]).wait()

        idx_start(u0 + 0, 0)
        idx_wait(u0 + 0, 0)
        gather_start(0)
        idx_start(u0 + 1, 1)

        def body2(t, carry):
            for b in range(2):
                u = t * 2 + 1 + b
                bf = 1 - b
                idx_wait(u0 + u, bf)
                gather_start(bf)
                gather_wait(b)
                idx_start(u0 + u + 1, b)

                @pl.when(u >= 3)
                def _():
                    out_wait(u0 + u - 3, b)
                select(b)
                out_start(u0 + u - 1, b)
            return carry

        lax.fori_loop(0, (u_per_w - 2) // 2, body2, 0)

        idx_wait(u0 + u_per_w - 1, 1)
        gather_start(1)
        gather_wait(0)
        out_wait(u0 + u_per_w - 4, 0)
        select(0)
        out_start(u0 + u_per_w - 2, 0)
        gather_wait(1)
        out_wait(u0 + u_per_w - 3, 1)
        select(1)
        out_start(u0 + u_per_w - 1, 1)
        out_wait(u0 + u_per_w - 2, 0)
        out_wait(u0 + u_per_w - 1, 1)

    return k(ids_t, table2)


def kernel(ids, table):
    b, s = ids.shape
    d = table.shape[1]
    ids_t = jnp.swapaxes(ids, 0, 1).astype(jnp.int32)
    table2 = jnp.pad(table, ((0, 0), (0, 128 - d)))
    out_t = _embed(ids_t, table2)
    return jnp.transpose(out_t, (2, 0, 1))
